# Initial kernel scaffold; baseline (speedup 1.0000x reference)
import jax
import jax.numpy as jnp
from jax.experimental import pallas as pl


def kernel(f, W_off, b_off, bn_gamma, bn_beta, W_dsc, b_dsc, gn_gamma, gn_beta):
    raise NotImplementedError("write your pallas kernel here")


# speedup vs baseline: 10.7285x; 10.7285x over previous
import functools

import jax
import jax.numpy as jnp
from jax.experimental import pallas as pl

H = 224
W = 224
CIN = 96
COUT = 96
K = 9
BH = 32
NB = H // BH
LIN = [-5.0 + 1.125 * k for k in range(K)]
DX = [-5, -4, -3, -2, -1, 0, 1, 2, 4]
F32 = jnp.float32


def _off_kernel(fz_ref, w_ref, b_ref, g_ref, be_ref, rows_ref):
    fz = fz_ref[...]
    acc = jnp.zeros((K, H * W), F32)
    for dy in range(3):
        for dx in range(3):
            sl = fz[:, dy:dy + H, dx:dx + W].reshape(CIN, H * W)
            acc = acc + jnp.dot(w_ref[dy, dx], sl,
                                preferred_element_type=F32)
    acc = acc + b_ref[...]
    mu = jnp.mean(acc, axis=1, keepdims=True)
    var = jnp.mean((acc - mu) ** 2, axis=1, keepdims=True)
    t = jnp.tanh((acc - mu) * jax.lax.rsqrt(var + 1e-5) * g_ref[...]
                 + be_ref[...])
    t = t.reshape(K, H, W)
    rows_ref[4, :, :] = jnp.zeros((H, W), F32)
    r = t[5]
    rows_ref[5, :, :] = r
    r = r + t[6]
    rows_ref[6, :, :] = r
    r = r + t[7]
    rows_ref[7, :, :] = r
    r = t[3]
    rows_ref[3, :, :] = r
    r = r + t[2]
    rows_ref[2, :, :] = r
    r = r + t[1]
    rows_ref[1, :, :] = r
    rows_ref[0, :, :] = t[0]
    rows_ref[8, :, :] = t[8]


def _samp_kernel(fpad_ref, roff_ref, wd_ref, bd_ref, out_ref, ps_ref, pq_ref):
    pid = pl.program_id(0)
    s = pid * BH
    fp = fpad_ref[:, pl.ds(s, BH + 7), :]
    roff = roff_ref[...]
    ii = (jax.lax.broadcasted_iota(jnp.int32, (BH, W), 0) + s).astype(F32)
    jj = jax.lax.broadcasted_iota(jnp.int32, (1, W), 1).astype(F32)
    acc = jnp.zeros((COUT, BH * W), F32)
    for k in range(K):
        xf = jj + LIN[k]
        x0f = jnp.clip(jj + float(DX[k]), 0.0, W - 1.0)
        x1f = jnp.clip(x0f + 1.0, 0.0, W - 1.0)
        wx0 = (x1f - xf)[None]
        wx1 = (xf - x0f)[None]
        c0 = 5 + DX[k]
        t1 = fp[:, :, c0 + 1:c0 + 1 + W]
        if DX[k] < 0:
            t1 = jnp.where((jj + float(DX[k]) < 0.0)[None], fp[:, :, 6:7], t1)
        g = wx0 * fp[:, :, c0:c0 + W] + wx1 * t1
        yf = ii + roff[k]
        fl = jnp.floor(yf)
        y0f = jnp.clip(fl, 0.0, H - 1.0)
        y1f = jnp.clip(y0f + 1.0, 0.0, H - 1.0)
        wy0 = y1f - yf
        wy1 = yf - y0f
        rel0 = y0f - ii
        rel1 = y1f - ii
        samp = jnp.zeros((CIN, BH, W), F32)
        for d in range(-3, 5):
            fd = float(d)
            cw = wy0 * (rel0 == fd).astype(F32) + wy1 * (rel1 == fd).astype(F32)
            samp = samp + cw[None] * g[:, d + 3:d + 3 + BH, :]
        acc = acc + jnp.dot(wd_ref[k], samp.reshape(CIN, BH * W),
                            preferred_element_type=F32)
    acc = acc + bd_ref[...]
    out = acc.reshape(COUT, BH, W)
    out_ref[...] = out
    ps_ref[pl.ds(pid, 1), :] = jnp.sum(acc, axis=1).reshape(1, COUT)
    pq_ref[pl.ds(pid, 1), :] = jnp.sum(acc * acc, axis=1).reshape(1, COUT)


def _gn_kernel(x_ref, sc_ref, sh_ref, o_ref):
    o_ref[...] = jnp.maximum(x_ref[...] * sc_ref[...] + sh_ref[...], 0.0)


@functools.partial(jax.jit, static_argnums=())
def kernel(f, W_off, b_off, bn_gamma, bn_beta, W_dsc, b_dsc, gn_gamma,
           gn_beta):
    f0 = f[0]
    fz = jnp.pad(f0, ((0, 0), (1, 1), (1, 1)))
    w9 = jnp.transpose(W_off[:K], (2, 3, 0, 1))
    rows = pl.pallas_call(
        _off_kernel,
        out_shape=jax.ShapeDtypeStruct((K, H, W), F32),
    )(fz, w9, b_off[:K, None], bn_gamma[:K, None], bn_beta[:K, None])

    fpad = jnp.pad(f0, ((0, 0), (3, 4), (5, 5)), mode='edge')
    wd = jnp.transpose(W_dsc[..., 0], (2, 0, 1))
    out, ps, pq = pl.pallas_call(
        _samp_kernel,
        grid=(NB,),
        in_specs=[
            pl.BlockSpec((CIN, H + 7, W + 10), lambda i: (0, 0, 0)),
            pl.BlockSpec((K, BH, W), lambda i: (0, i, 0)),
            pl.BlockSpec((K, COUT, CIN), lambda i: (0, 0, 0)),
            pl.BlockSpec((COUT, 1), lambda i: (0, 0)),
        ],
        out_specs=[
            pl.BlockSpec((COUT, BH, W), lambda i: (0, i, 0)),
            pl.BlockSpec((NB, COUT), lambda i: (0, 0)),
            pl.BlockSpec((NB, COUT), lambda i: (0, 0)),
        ],
        out_shape=[
            jax.ShapeDtypeStruct((COUT, H, W), F32),
            jax.ShapeDtypeStruct((NB, COUT), F32),
            jax.ShapeDtypeStruct((NB, COUT), F32),
        ],
    )(fpad, rows, wd, b_dsc[:, None])

    grp = COUT // (COUT // 4)
    npix = grp * H * W
    csum = jnp.sum(ps, axis=0).reshape(-1, grp)
    csq = jnp.sum(pq, axis=0).reshape(-1, grp)
    gmu = jnp.sum(csum, axis=1) / npix
    gvar = jnp.maximum(jnp.sum(csq, axis=1) / npix - gmu * gmu, 0.0)
    inv = jax.lax.rsqrt(gvar + 1e-5)
    scale = gn_gamma * jnp.repeat(inv, grp)
    shift = gn_beta - jnp.repeat(gmu, grp) * scale

    xn = pl.pallas_call(
        _gn_kernel,
        grid=(NB,),
        in_specs=[
            pl.BlockSpec((COUT, BH, W), lambda i: (0, i, 0)),
            pl.BlockSpec((COUT, 1, 1), lambda i: (0, 0, 0)),
            pl.BlockSpec((COUT, 1, 1), lambda i: (0, 0, 0)),
        ],
        out_specs=pl.BlockSpec((COUT, BH, W), lambda i: (0, i, 0)),
        out_shape=jax.ShapeDtypeStruct((COUT, H, W), F32),
    )(out, scale[:, None, None], shift[:, None, None])
    return xn[None]


# speedup vs baseline: 10.9757x; 1.0230x over previous
import functools

import jax
import jax.numpy as jnp
from jax.experimental import pallas as pl

H = 224
W = 224
CIN = 96
COUT = 96
K = 9
BH = 56
NB = H // BH
LIN = [-5.0 + 1.125 * k for k in range(K)]
DX = [-5, -4, -3, -2, -1, 0, 1, 2, 4]
F32 = jnp.float32


def _off_kernel(fz_ref, w_ref, b_ref, g_ref, be_ref, rows_ref):
    fz = fz_ref[...]
    acc = jnp.zeros((K, H * W), F32)
    for dy in range(3):
        for dx in range(3):
            sl = fz[:, dy:dy + H, dx:dx + W].reshape(CIN, H * W)
            acc = acc + jnp.dot(w_ref[dy, dx], sl,
                                preferred_element_type=F32)
    acc = acc + b_ref[...]
    mu = jnp.mean(acc, axis=1, keepdims=True)
    var = jnp.mean((acc - mu) ** 2, axis=1, keepdims=True)
    t = jnp.tanh((acc - mu) * jax.lax.rsqrt(var + 1e-5) * g_ref[...]
                 + be_ref[...])
    t = t.reshape(K, H, W)
    rows_ref[4, :, :] = jnp.zeros((H, W), F32)
    r = t[5]
    rows_ref[5, :, :] = r
    r = r + t[6]
    rows_ref[6, :, :] = r
    r = r + t[7]
    rows_ref[7, :, :] = r
    r = t[3]
    rows_ref[3, :, :] = r
    r = r + t[2]
    rows_ref[2, :, :] = r
    r = r + t[1]
    rows_ref[1, :, :] = r
    rows_ref[0, :, :] = t[0]
    rows_ref[8, :, :] = t[8]


def _samp_kernel(fpad_ref, roff_ref, wd_ref, bd_ref, out_ref, ps_ref, pq_ref):
    pid = pl.program_id(0)
    s = pid * BH
    fp = fpad_ref[:, pl.ds(s, BH + 7), :]
    roff = roff_ref[...]
    ii = (jax.lax.broadcasted_iota(jnp.int32, (BH, W), 0) + s).astype(F32)
    jj = jax.lax.broadcasted_iota(jnp.int32, (1, W), 1).astype(F32)
    acc = jnp.zeros((COUT, BH * W), F32)
    for k in range(K):
        xf = jj + LIN[k]
        x0f = jnp.clip(jj + float(DX[k]), 0.0, W - 1.0)
        x1f = jnp.clip(x0f + 1.0, 0.0, W - 1.0)
        wx0 = (x1f - xf)[None]
        wx1 = (xf - x0f)[None]
        c0 = 5 + DX[k]
        t1 = fp[:, :, c0 + 1:c0 + 1 + W]
        if DX[k] < 0:
            t1 = jnp.where((jj + float(DX[k]) < 0.0)[None], fp[:, :, 6:7], t1)
        g = wx0 * fp[:, :, c0:c0 + W] + wx1 * t1
        yf = ii + roff[k]
        fl = jnp.floor(yf)
        y0f = jnp.clip(fl, 0.0, H - 1.0)
        y1f = jnp.clip(y0f + 1.0, 0.0, H - 1.0)
        wy0 = y1f - yf
        wy1 = yf - y0f
        rel0 = y0f - ii
        rel1 = y1f - ii
        samp = jnp.zeros((CIN, BH, W), F32)
        for d in range(-3, 5):
            fd = float(d)
            cw = wy0 * (rel0 == fd).astype(F32) + wy1 * (rel1 == fd).astype(F32)
            samp = samp + cw[None] * g[:, d + 3:d + 3 + BH, :]
        acc = acc + jnp.dot(wd_ref[k], samp.reshape(CIN, BH * W),
                            preferred_element_type=F32)
    acc = acc + bd_ref[...]
    out = acc.reshape(COUT, BH, W)
    out_ref[...] = out
    ps_ref[pl.ds(pid, 1), :] = jnp.sum(acc, axis=1).reshape(1, COUT)
    pq_ref[pl.ds(pid, 1), :] = jnp.sum(acc * acc, axis=1).reshape(1, COUT)


def _gn_kernel(x_ref, sc_ref, sh_ref, o_ref):
    o_ref[...] = jnp.maximum(x_ref[...] * sc_ref[...] + sh_ref[...], 0.0)


@functools.partial(jax.jit, static_argnums=())
def kernel(f, W_off, b_off, bn_gamma, bn_beta, W_dsc, b_dsc, gn_gamma,
           gn_beta):
    f0 = f[0]
    fz = jnp.pad(f0, ((0, 0), (1, 1), (1, 1)))
    w9 = jnp.transpose(W_off[:K], (2, 3, 0, 1))
    rows = pl.pallas_call(
        _off_kernel,
        out_shape=jax.ShapeDtypeStruct((K, H, W), F32),
    )(fz, w9, b_off[:K, None], bn_gamma[:K, None], bn_beta[:K, None])

    fpad = jnp.pad(f0, ((0, 0), (3, 4), (5, 5)), mode='edge')
    wd = jnp.transpose(W_dsc[..., 0], (2, 0, 1))
    out, ps, pq = pl.pallas_call(
        _samp_kernel,
        grid=(NB,),
        in_specs=[
            pl.BlockSpec((CIN, H + 7, W + 10), lambda i: (0, 0, 0)),
            pl.BlockSpec((K, BH, W), lambda i: (0, i, 0)),
            pl.BlockSpec((K, COUT, CIN), lambda i: (0, 0, 0)),
            pl.BlockSpec((COUT, 1), lambda i: (0, 0)),
        ],
        out_specs=[
            pl.BlockSpec((COUT, BH, W), lambda i: (0, i, 0)),
            pl.BlockSpec((NB, COUT), lambda i: (0, 0)),
            pl.BlockSpec((NB, COUT), lambda i: (0, 0)),
        ],
        out_shape=[
            jax.ShapeDtypeStruct((COUT, H, W), F32),
            jax.ShapeDtypeStruct((NB, COUT), F32),
            jax.ShapeDtypeStruct((NB, COUT), F32),
        ],
    )(fpad, rows, wd, b_dsc[:, None])

    grp = COUT // (COUT // 4)
    npix = grp * H * W
    csum = jnp.sum(ps, axis=0).reshape(-1, grp)
    csq = jnp.sum(pq, axis=0).reshape(-1, grp)
    gmu = jnp.sum(csum, axis=1) / npix
    gvar = jnp.maximum(jnp.sum(csq, axis=1) / npix - gmu * gmu, 0.0)
    inv = jax.lax.rsqrt(gvar + 1e-5)
    scale = gn_gamma * jnp.repeat(inv, grp)
    shift = gn_beta - jnp.repeat(gmu, grp) * scale

    xn = pl.pallas_call(
        _gn_kernel,
        grid=(NB,),
        in_specs=[
            pl.BlockSpec((COUT, BH, W), lambda i: (0, i, 0)),
            pl.BlockSpec((COUT, 1, 1), lambda i: (0, 0, 0)),
            pl.BlockSpec((COUT, 1, 1), lambda i: (0, 0, 0)),
        ],
        out_specs=pl.BlockSpec((COUT, BH, W), lambda i: (0, i, 0)),
        out_shape=jax.ShapeDtypeStruct((COUT, H, W), F32),
    )(out, scale[:, None, None], shift[:, None, None])
    return xn[None]
